# baseline (speedup 1.0000x reference)
import functools

import jax
import jax.numpy as jnp
from jax import lax
from jax.experimental import pallas as pl
from jax.experimental.pallas import tpu as pltpu
from jax.experimental.pallas import tpu_sc as plsc

NUM_EMBEDDINGS = 1000000
EMBED_DIM = 3
B_ROWS = 16384
B_COLS = 26
TOTAL = B_ROWS * B_COLS
TABLE_WORDS = NUM_EMBEDDINGS * EMBED_DIM
NUM_WORKERS = 32
BPW = B_ROWS // NUM_WORKERS
PLANE = B_COLS * BPW
SLAB = EMBED_DIM * PLANE
LANES = 16
GRPS = PLANE // LANES


def _gather_body(tab_hbm, idx_hbm, out_hbm, idx_v, widx_v, rows_v, sem, gsem):
    wid = lax.axis_index("s") * 2 + lax.axis_index("c")
    b0 = wid * BPW

    for j in range(B_COLS):
        pltpu.async_copy(
            idx_hbm.at[j, pl.ds(b0, BPW)], idx_v.at[pl.ds(j * BPW, BPW)], sem
        )
    for j in range(B_COLS):
        pltpu.make_async_copy(
            idx_hbm.at[j, pl.ds(b0, BPW)], idx_v.at[pl.ds(j * BPW, BPW)], sem
        ).wait()

    for c in range(EMBED_DIM):
        def grp(g, carry, c=c):
            s = g * LANES
            widx_v[pl.ds(c * PLANE + s, LANES)] = (
                idx_v[pl.ds(s, LANES)] + c * NUM_EMBEDDINGS
            )
            return carry

        lax.fori_loop(0, GRPS, grp, 0)

    pltpu.async_copy(tab_hbm.at[widx_v], rows_v, gsem).wait()
    pltpu.sync_copy(rows_v, out_hbm.at[wid])


_gather_call = pl.kernel(
    _gather_body,
    out_type=jax.ShapeDtypeStruct((NUM_WORKERS, SLAB), jnp.float32),
    mesh=plsc.VectorSubcoreMesh(core_axis_name="c", subcore_axis_name="s"),
    scratch_types=[
        pltpu.VMEM((PLANE,), jnp.int32),
        pltpu.VMEM((SLAB,), jnp.int32),
        pltpu.VMEM((SLAB,), jnp.float32),
        pltpu.SemaphoreType.DMA,
        pltpu.SemaphoreType.DMA,
    ],
    compiler_params=pltpu.CompilerParams(
        use_tc_tiling_on_sc=False, needs_layout_passes=False
    ),
)


@jax.jit
def kernel(inputs, weight):
    flat_w = weight.T.reshape(TABLE_WORDS)
    idx_cm = inputs.T.astype(jnp.int32)
    out = _gather_call(flat_w, idx_cm)
    out = out.reshape(NUM_WORKERS, EMBED_DIM, B_COLS, BPW)
    out = out.transpose(1, 2, 0, 3).reshape(EMBED_DIM, B_COLS, B_ROWS)
    return out.transpose(2, 1, 0)


# speedup vs baseline: 1.1840x; 1.1840x over previous
import functools

import jax
import jax.numpy as jnp
from jax import lax
from jax.experimental import pallas as pl
from jax.experimental.pallas import tpu as pltpu
from jax.experimental.pallas import tpu_sc as plsc

NUM_EMBEDDINGS = 1000000
EMBED_DIM = 3
B_ROWS = 16384
B_COLS = 26
TOTAL = B_ROWS * B_COLS
TABLE_WORDS = NUM_EMBEDDINGS * EMBED_DIM
NUM_WORKERS = 32
BPW = B_ROWS // NUM_WORKERS
ROWS = EMBED_DIM * B_COLS
LANES = 16
GRP = BPW // LANES


def _gather_body(tab_hbm, idx_hbm, out_hbm, idx_v, widx_v, rows_v, sem, gsem):
    wid = lax.axis_index("s") * 2 + lax.axis_index("c")
    b0 = wid * BPW

    pltpu.sync_copy(idx_hbm.at[:, pl.ds(b0, BPW)], idx_v)

    def expand_fire(r, carry):
        j = r % B_COLS
        plane = (r // B_COLS) * NUM_EMBEDDINGS

        def grp(g, c2):
            widx_v[r, pl.ds(g * LANES, LANES)] = (
                idx_v[j, pl.ds(g * LANES, LANES)] + plane
            )
            return c2

        lax.fori_loop(0, GRP, grp, 0)
        pltpu.async_copy(tab_hbm.at[widx_v.at[r]], rows_v.at[r], gsem)
        return carry

    lax.fori_loop(0, ROWS, expand_fire, 0)

    def drain(r, carry):
        pltpu.make_async_copy(
            tab_hbm.at[widx_v.at[r]], rows_v.at[r], gsem
        ).wait()
        return carry

    lax.fori_loop(0, ROWS, drain, 0)

    pltpu.sync_copy(rows_v, out_hbm.at[:, pl.ds(b0, BPW)])


_gather_call = pl.kernel(
    _gather_body,
    out_type=jax.ShapeDtypeStruct((ROWS, B_ROWS), jnp.float32),
    mesh=plsc.VectorSubcoreMesh(core_axis_name="c", subcore_axis_name="s"),
    scratch_types=[
        pltpu.VMEM((B_COLS, BPW), jnp.int32),
        pltpu.VMEM((ROWS, BPW), jnp.int32),
        pltpu.VMEM((ROWS, BPW), jnp.float32),
        pltpu.SemaphoreType.DMA,
        pltpu.SemaphoreType.DMA,
    ],
    compiler_params=pltpu.CompilerParams(
        use_tc_tiling_on_sc=False, needs_layout_passes=False
    ),
)


@jax.jit
def kernel(inputs, weight):
    flat_w = weight.T.reshape(TABLE_WORDS)
    idx_cm = inputs.T.astype(jnp.int32)
    out = _gather_call(flat_w, idx_cm)
    return out.reshape(EMBED_DIM, B_COLS, B_ROWS).transpose(2, 1, 0)
